# Initial kernel scaffold; baseline (speedup 1.0000x reference)
import jax
import jax.numpy as jnp
from jax.experimental import pallas as pl


def kernel(h, edge_index, Wm, bm, Ws, bs, g1, beta1, W1, bf1, W2, bf2, g2, beta2):
    raise NotImplementedError("write your pallas kernel here")


# speedup vs baseline: 3.9718x; 3.9718x over previous
import functools

import jax
import jax.numpy as jnp
from jax import lax
from jax.experimental import pallas as pl
from jax.experimental.pallas import tpu as pltpu
from jax.experimental.pallas import tpu_sc as plsc

N = 10000
D = 256
E = 160000

NUM_TILES = 16
EDGES_PER_TILE = E // NUM_TILES
CHUNK = 80
CHUNKS_PER_TILE = EDGES_PER_TILE // CHUNK
TW = 128
NPAD = 10240
ROWS_PER_TILE = NPAD // NUM_TILES
ZROWS = 128
DEGROWS = NPAD // 128


def _sc_aggregate(t0, t1, src_r, dst_r):
  mesh = plsc.VectorSubcoreMesh(core_axis_name="c", subcore_axis_name="s")

  @functools.partial(
      pl.kernel,
      mesh=mesh,
      compiler_params=pltpu.CompilerParams(needs_layout_passes=False),
      out_type=[
          jax.ShapeDtypeStruct((NPAD, TW), jnp.float32),
          jax.ShapeDtypeStruct((NPAD, TW), jnp.float32),
          jax.ShapeDtypeStruct((NPAD,), jnp.float32),
      ],
      scratch_types=[
          pltpu.VMEM((CHUNK,), jnp.int32),
          pltpu.VMEM((CHUNK,), jnp.int32),
          pltpu.VMEM((CHUNK, TW), jnp.float32),
          pltpu.VMEM((ZROWS, TW), jnp.float32),
          pltpu.VMEM((NPAD,), jnp.float32),
          pltpu.VMEM((ROWS_PER_TILE,), jnp.float32),
          pltpu.VMEM((ROWS_PER_TILE,), jnp.float32),
          pltpu.VMEM_SHARED((NPAD, TW), jnp.float32),
          pltpu.VMEM_SHARED((NUM_TILES * NPAD,), jnp.float32),
          pltpu.SemaphoreType.DMA,
      ],
  )
  def agg_kernel(t0_hbm, t1_hbm, src_hbm, dst_hbm,
                 out0_hbm, out1_hbm, deg_hbm,
                 src_v, dst_v, rows_v, zero_v, deg_v, tmp_v, red_v, acc_sh,
                 parts_sh, sem):
    c = lax.axis_index("c")
    s = lax.axis_index("s")

    zvec = jnp.zeros((16,), jnp.float32)
    ones16 = jnp.ones((16,), jnp.float32)

    def zrow(r, _):
      for j in range(TW // 16):
        zero_v[r, pl.ds(j * 16, 16)] = zvec
      return _

    lax.fori_loop(0, ZROWS, zrow, 0)

    def zdeg(r, _):
      deg_v[pl.ds(r * 16, 16)] = zvec
      return _

    lax.fori_loop(0, NPAD // 16, zdeg, 0)

    for z in range(ROWS_PER_TILE // ZROWS):
      pltpu.sync_copy(
          zero_v, acc_sh.at[pl.ds(s * ROWS_PER_TILE + z * ZROWS, ZROWS)])

    plsc.subcore_barrier()

    def run_core(t_hbm, out_hbm, count_deg):
      def body(j, _):
        row = s * CHUNKS_PER_TILE + j
        pltpu.sync_copy(src_hbm.at[row], src_v)
        pltpu.sync_copy(dst_hbm.at[row], dst_v)
        pltpu.async_copy(t_hbm.at[src_v], rows_v, sem).wait()
        pltpu.sync_copy(rows_v, acc_sh.at[dst_v], add=True)
        if count_deg:
          for g in range(CHUNK // 16):
            dvec = dst_v[pl.ds(g * 16, 16)]
            plsc.addupdate_scatter(deg_v, [dvec], ones16)
        return _

      lax.fori_loop(0, CHUNKS_PER_TILE, body, 0)
      if count_deg:
        pltpu.sync_copy(deg_v, parts_sh.at[pl.ds(s * NPAD, NPAD)])
      plsc.subcore_barrier()
      base = s * ROWS_PER_TILE
      pltpu.sync_copy(acc_sh.at[pl.ds(base, ROWS_PER_TILE)],
                      out_hbm.at[pl.ds(base, ROWS_PER_TILE)])

    @pl.when(c == 0)
    def _():
      run_core(t0_hbm, out0_hbm, True)
      base = s * ROWS_PER_TILE

      def zred(r, _):
        red_v[pl.ds(r * 16, 16)] = zvec
        return _

      lax.fori_loop(0, ROWS_PER_TILE // 16, zred, 0)
      for p in range(NUM_TILES):
        pltpu.sync_copy(parts_sh.at[pl.ds(p * NPAD + base, ROWS_PER_TILE)],
                        tmp_v)
        def radd(r, _):
          red_v[pl.ds(r * 16, 16)] = (red_v[pl.ds(r * 16, 16)]
                                      + tmp_v[pl.ds(r * 16, 16)])
          return _
        lax.fori_loop(0, ROWS_PER_TILE // 16, radd, 0)
      pltpu.sync_copy(red_v, deg_hbm.at[pl.ds(base, ROWS_PER_TILE)])

    @pl.when(c == 1)
    def _():
      run_core(t1_hbm, out1_hbm, False)

  return agg_kernel(t0, t1, src_r, dst_r)


def _layernorm(x, g, b, eps=1e-5):
  mu = jnp.mean(x, axis=-1, keepdims=True)
  var = jnp.mean((x - mu) ** 2, axis=-1, keepdims=True)
  return (x - mu) * jax.lax.rsqrt(var + eps) * g + b


ROW_BLK = 1000


def _dense_body(h_ref, a0_ref, a1_ref, deg_ref, A0_ref, A1_ref, bm_ref,
                WsT_ref, bs_ref, W1T_ref, bf1_ref, W2T_ref, bf2_ref,
                g1_ref, b1_ref, g2_ref, b2_ref, out_ref):
  h = h_ref[...]
  dot = functools.partial(jnp.dot, preferred_element_type=jnp.float32)
  pre = dot(a0_ref[...], A0_ref[...]) + dot(a1_ref[...], A1_ref[...])
  pre = pre + bm_ref[...] * deg_ref[...]
  agg = pre / jnp.maximum(deg_ref[...], 1.0)
  x = h + jnp.maximum(dot(h, WsT_ref[...]) + bs_ref[...] + agg, 0.0)
  h1 = _layernorm(x, g1_ref[...], b1_ref[...])
  hid = dot(h1, W1T_ref[...]) + bf1_ref[...]
  hid = hid * 0.5 * (1.0 + lax.erf(hid * (2.0 ** -0.5)))
  ffn = dot(hid, W2T_ref[...]) + bf2_ref[...]
  out_ref[...] = _layernorm(h1 + ffn, g2_ref[...], b2_ref[...])


def _tc_dense(h, acc0, acc1, deg, A0, A1, bm, WsT, bs, W1T, bf1, W2T, bf2,
              g1, b1, g2, b2):
  grid = (N // ROW_BLK,)
  row_spec = lambda w: pl.BlockSpec((ROW_BLK, w), lambda i: (i, 0))
  full = lambda a: pl.BlockSpec(a.shape, lambda i: (0,) * a.ndim)
  return pl.pallas_call(
      _dense_body,
      grid=grid,
      in_specs=[
          row_spec(D), row_spec(TW), row_spec(TW), row_spec(1),
          full(A0), full(A1), full(bm), full(WsT), full(bs),
          full(W1T), full(bf1), full(W2T), full(bf2),
          full(g1), full(b1), full(g2), full(b2),
      ],
      out_specs=row_spec(D),
      out_shape=jax.ShapeDtypeStruct((N, D), jnp.float32),
  )(h, acc0, acc1, deg, A0, A1, bm, WsT, bs, W1T, bf1, W2T, bf2,
    g1, b1, g2, b2)


@jax.jit
def kernel(h, edge_index, Wm, bm, Ws, bs, g1, beta1, W1, bf1, W2, bf2,
           g2, beta2):
  src = edge_index[0].astype(jnp.int32).reshape(E // CHUNK, CHUNK)
  dst = edge_index[1].astype(jnp.int32).reshape(E // CHUNK, CHUNK)

  t0 = h[:, : D // 2]
  t1 = h[:, D // 2 :]

  acc0, acc1, degcnt = _sc_aggregate(t0, t1, src, dst)
  deg = degcnt.reshape(NPAD, 1)

  WmT = Wm.T
  A0 = WmT[: D // 2]
  A1 = WmT[D // 2 :]

  return _tc_dense(
      h, acc0, acc1, deg, A0, A1, bm[None, :], Ws.T, bs[None, :],
      W1.T, bf1[None, :], W2.T, bf2[None, :], g1[None, :], beta1[None, :],
      g2[None, :], beta2[None, :])


# speedup vs baseline: 5.8177x; 1.4648x over previous
import functools

import jax
import jax.numpy as jnp
from jax import lax
from jax.experimental import pallas as pl
from jax.experimental.pallas import tpu as pltpu
from jax.experimental.pallas import tpu_sc as plsc

N = 10000
D = 256
E = 160000

NUM_TILES = 16
EDGES_PER_TILE = E // NUM_TILES
CHUNK = 80
CHUNKS_PER_TILE = EDGES_PER_TILE // CHUNK
TW = 128
NPAD = 10240
ROWS_PER_TILE = NPAD // NUM_TILES
ZROWS = 32
DEGROWS = NPAD // 128


def _sc_aggregate(t0, t1, src_r, dst_r):
  mesh = plsc.VectorSubcoreMesh(core_axis_name="c", subcore_axis_name="s")

  @functools.partial(
      pl.kernel,
      mesh=mesh,
      compiler_params=pltpu.CompilerParams(needs_layout_passes=False),
      out_type=[
          jax.ShapeDtypeStruct((NPAD, TW), jnp.float32),
          jax.ShapeDtypeStruct((NPAD, TW), jnp.float32),
          jax.ShapeDtypeStruct((NPAD,), jnp.float32),
      ],
      scratch_types=[
          pltpu.VMEM((CHUNK,), jnp.int32),
          pltpu.VMEM((CHUNK,), jnp.int32),
          pltpu.VMEM((CHUNK,), jnp.int32),
          pltpu.VMEM((CHUNK,), jnp.int32),
          pltpu.VMEM((CHUNK, TW), jnp.float32),
          pltpu.VMEM((CHUNK, TW), jnp.float32),
          pltpu.VMEM((ZROWS, TW), jnp.float32),
          pltpu.VMEM((NPAD,), jnp.float32),
          pltpu.VMEM((ROWS_PER_TILE,), jnp.float32),
          pltpu.VMEM((ROWS_PER_TILE,), jnp.float32),
          pltpu.VMEM_SHARED((NPAD, TW), jnp.float32),
          pltpu.VMEM_SHARED((NUM_TILES * NPAD,), jnp.float32),
          pltpu.SemaphoreType.DMA,
          pltpu.SemaphoreType.DMA,
      ],
  )
  def agg_kernel(t0_hbm, t1_hbm, src_hbm, dst_hbm,
                 out0_hbm, out1_hbm, deg_hbm,
                 src0_v, dst0_v, src1_v, dst1_v, rows0_v, rows1_v, zero_v,
                 deg_v, tmp_v, red_v, acc_sh, parts_sh, sem0, sem1):
    c = lax.axis_index("c")
    s = lax.axis_index("s")

    zvec = jnp.zeros((16,), jnp.float32)
    ones16 = jnp.ones((16,), jnp.float32)

    def zrow(r, _):
      for j in range(TW // 16):
        zero_v[r, pl.ds(j * 16, 16)] = zvec
      return _

    lax.fori_loop(0, ZROWS, zrow, 0)

    def zdeg(r, _):
      deg_v[pl.ds(r * 16, 16)] = zvec
      return _

    lax.fori_loop(0, NPAD // 16, zdeg, 0)

    for z in range(ROWS_PER_TILE // ZROWS):
      pltpu.sync_copy(
          zero_v, acc_sh.at[pl.ds(s * ROWS_PER_TILE + z * ZROWS, ZROWS)])

    plsc.subcore_barrier()

    def run_core(t_hbm, out_hbm, count_deg):
      bufs = ((src0_v, dst0_v, rows0_v, sem0),
              (src1_v, dst1_v, rows1_v, sem1))

      def fire(j, b):
        srcb, dstb, rows, sem = bufs[b]
        row = s * CHUNKS_PER_TILE + j
        pltpu.sync_copy(src_hbm.at[row], srcb)
        pltpu.sync_copy(dst_hbm.at[row], dstb)
        pltpu.async_copy(t_hbm.at[srcb], rows, sem)

      def drain(j, b):
        srcb, dstb, rows, sem = bufs[b]
        pltpu.make_async_copy(t_hbm.at[srcb], rows, sem).wait()
        pltpu.sync_copy(rows, acc_sh.at[dstb], add=True)
        if count_deg:
          for g in range(CHUNK // 16):
            dvec = dstb[pl.ds(g * 16, 16)]
            plsc.addupdate_scatter(deg_v, [dvec], ones16)

      fire(0, 0)

      def body(j2, _):
        j = 2 * j2
        fire(j + 1, 1)
        drain(j, 0)
        fire(j + 2, 0)
        drain(j + 1, 1)
        return _

      lax.fori_loop(0, (CHUNKS_PER_TILE - 1) // 2, body, 0)
      drain(CHUNKS_PER_TILE - 1, 0)
      if count_deg:
        pltpu.sync_copy(deg_v, parts_sh.at[pl.ds(s * NPAD, NPAD)])
      plsc.subcore_barrier()
      base = s * ROWS_PER_TILE
      pltpu.sync_copy(acc_sh.at[pl.ds(base, ROWS_PER_TILE)],
                      out_hbm.at[pl.ds(base, ROWS_PER_TILE)])

    @pl.when(c == 0)
    def _():
      run_core(t0_hbm, out0_hbm, True)
      base = s * ROWS_PER_TILE

      def zred(r, _):
        red_v[pl.ds(r * 16, 16)] = zvec
        return _

      lax.fori_loop(0, ROWS_PER_TILE // 16, zred, 0)
      for p in range(NUM_TILES):
        pltpu.sync_copy(parts_sh.at[pl.ds(p * NPAD + base, ROWS_PER_TILE)],
                        tmp_v)
        def radd(r, _):
          red_v[pl.ds(r * 16, 16)] = (red_v[pl.ds(r * 16, 16)]
                                      + tmp_v[pl.ds(r * 16, 16)])
          return _
        lax.fori_loop(0, ROWS_PER_TILE // 16, radd, 0)
      pltpu.sync_copy(red_v, deg_hbm.at[pl.ds(base, ROWS_PER_TILE)])

    @pl.when(c == 1)
    def _():
      run_core(t1_hbm, out1_hbm, False)

  return agg_kernel(t0, t1, src_r, dst_r)


def _layernorm(x, g, b, eps=1e-5):
  mu = jnp.mean(x, axis=-1, keepdims=True)
  var = jnp.mean((x - mu) ** 2, axis=-1, keepdims=True)
  return (x - mu) * jax.lax.rsqrt(var + eps) * g + b


ROW_BLK = 1000


def _dense_body(h_ref, a0_ref, a1_ref, deg_ref, A0_ref, A1_ref, bm_ref,
                WsT_ref, bs_ref, W1T_ref, bf1_ref, W2T_ref, bf2_ref,
                g1_ref, b1_ref, g2_ref, b2_ref, out_ref):
  h = h_ref[...]
  dot = functools.partial(jnp.dot, preferred_element_type=jnp.float32)
  pre = dot(a0_ref[...], A0_ref[...]) + dot(a1_ref[...], A1_ref[...])
  pre = pre + bm_ref[...] * deg_ref[...]
  agg = pre / jnp.maximum(deg_ref[...], 1.0)
  x = h + jnp.maximum(dot(h, WsT_ref[...]) + bs_ref[...] + agg, 0.0)
  h1 = _layernorm(x, g1_ref[...], b1_ref[...])
  hid = dot(h1, W1T_ref[...]) + bf1_ref[...]
  hid = hid * 0.5 * (1.0 + lax.erf(hid * (2.0 ** -0.5)))
  ffn = dot(hid, W2T_ref[...]) + bf2_ref[...]
  out_ref[...] = _layernorm(h1 + ffn, g2_ref[...], b2_ref[...])


def _tc_dense(h, acc0, acc1, deg, A0, A1, bm, WsT, bs, W1T, bf1, W2T, bf2,
              g1, b1, g2, b2):
  grid = (N // ROW_BLK,)
  row_spec = lambda w: pl.BlockSpec((ROW_BLK, w), lambda i: (i, 0))
  full = lambda a: pl.BlockSpec(a.shape, lambda i: (0,) * a.ndim)
  return pl.pallas_call(
      _dense_body,
      grid=grid,
      in_specs=[
          row_spec(D), row_spec(TW), row_spec(TW), row_spec(1),
          full(A0), full(A1), full(bm), full(WsT), full(bs),
          full(W1T), full(bf1), full(W2T), full(bf2),
          full(g1), full(b1), full(g2), full(b2),
      ],
      out_specs=row_spec(D),
      out_shape=jax.ShapeDtypeStruct((N, D), jnp.float32),
  )(h, acc0, acc1, deg, A0, A1, bm, WsT, bs, W1T, bf1, W2T, bf2,
    g1, b1, g2, b2)


@jax.jit
def kernel(h, edge_index, Wm, bm, Ws, bs, g1, beta1, W1, bf1, W2, bf2,
           g2, beta2):
  src = edge_index[0].astype(jnp.int32).reshape(E // CHUNK, CHUNK)
  dst = edge_index[1].astype(jnp.int32).reshape(E // CHUNK, CHUNK)

  t0 = h[:, : D // 2]
  t1 = h[:, D // 2 :]

  acc0, acc1, degcnt = _sc_aggregate(t0, t1, src, dst)
  deg = degcnt.reshape(NPAD, 1)

  WmT = Wm.T
  A0 = WmT[: D // 2]
  A1 = WmT[D // 2 :]

  return _tc_dense(
      h, acc0, acc1, deg, A0, A1, bm[None, :], Ws.T, bs[None, :],
      W1.T, bf1[None, :], W2.T, bf2[None, :], g1[None, :], beta1[None, :],
      g2[None, :], beta2[None, :])
